# baseline (speedup 1.0000x reference)
import jax
import jax.numpy as jnp
from jax import lax
from jax.experimental import pallas as pl
from jax.experimental.pallas import tpu as pltpu
from jax.experimental.pallas import tpu_sc as plsc

B, L, D = 4096, 200, 64
NC, NS = 2, 16
NW = NC * NS
TOK = B * L
TPW = TOK // NW
T = 400
NCH = TPW // T
VPD = D // 16
NBUF = 2


def _body(emb_hbm, x_hbm, pos_hbm, out_hbm,
          idx_all, pos_v, rows0, rows1, isem, g0, g1, o0, o1):
    rows = (rows0, rows1)
    gsem = (g0, g1)
    osem = (o0, o1)
    wid = lax.axis_index("s") * NC + lax.axis_index("c")
    base_w = wid * TPW

    idx_cp = pltpu.async_copy(x_hbm.at[pl.ds(base_w, TPW)], idx_all, isem)
    pltpu.sync_copy(pos_hbm.at[pl.ds(0, L)], pos_v)
    idx_cp.wait()

    def idx_slice(c):
        return idx_all.at[pl.ds(c * T, T)]

    def gather(c, b):
        pltpu.async_copy(emb_hbm.at[idx_slice(c)], rows[b], gsem[b])

    def gather_wait(c, b):
        pltpu.make_async_copy(emb_hbm.at[idx_slice(c)], rows[b],
                              gsem[b]).wait()

    def out_copy(c, b):
        pltpu.async_copy(rows[b], out_hbm.at[pl.ds(base_w + c * T, T)],
                         osem[b])

    def out_wait(c, b):
        pltpu.make_async_copy(rows[b], out_hbm.at[pl.ds(base_w + c * T, T)],
                              osem[b]).wait()

    gather(0, 0)

    @pl.loop(0, NCH, step=NBUF)
    def _outer(i0):
        for b in range(NBUF):
            c = i0 + b
            o = 1 - b

            @pl.when(c + 1 < NCH)
            def _launch_next():
                @pl.when(c >= 1)
                def _drain_prev_out():
                    out_wait(c - 1, o)
                gather(c + 1, o)

            gather_wait(c, b)

            @pl.loop(0, L, unroll=2)
            def _add(j):
                for v in range(VPD):
                    p = pos_v[j, pl.ds(v * 16, 16)]
                    for r in range(T // L):
                        t = r * L + j
                        rows[b][t, pl.ds(v * 16, 16)] = (
                            rows[b][t, pl.ds(v * 16, 16)] + p
                        )

            out_copy(c, b)

    out_wait(NCH - 2, (NCH - 2) % NBUF)
    out_wait(NCH - 1, (NCH - 1) % NBUF)


@jax.jit
def kernel(x, embed_table, pos_table):
    x_flat = x.reshape(TOK).astype(jnp.int32)
    mesh = plsc.VectorSubcoreMesh(core_axis_name="c", subcore_axis_name="s",
                                  num_cores=NC, num_subcores=NS)
    out = pl.kernel(
        _body,
        out_type=jax.ShapeDtypeStruct((TOK, D), jnp.float32),
        mesh=mesh,
        compiler_params=pltpu.CompilerParams(use_tc_tiling_on_sc=False),
        scratch_types=[
            pltpu.VMEM((TPW,), jnp.int32),
            pltpu.VMEM((L, D), jnp.float32),
            pltpu.VMEM((T, D), jnp.float32),
            pltpu.VMEM((T, D), jnp.float32),
            pltpu.SemaphoreType.DMA,
            pltpu.SemaphoreType.DMA,
            pltpu.SemaphoreType.DMA,
            pltpu.SemaphoreType.DMA,
            pltpu.SemaphoreType.DMA,
        ],
    )(embed_table, x_flat, pos_table)
    return out.reshape(B, L, D)
